# baseline (speedup 1.0000x reference)
import jax
import jax.numpy as jnp
from jax import lax
from jax.experimental import pallas as pl
from jax.experimental.pallas import tpu as pltpu
from jax.experimental.pallas import tpu_sc as plsc

N_NODES = 10000
EMB = 128
NUM_LAYER = 5
NUM_GRAPHS = 512
BN_EPS = 1e-5

NC = 2
NS = 16
NW = NC * NS

NPAD = 10240
PAD_ROW = N_NODES
NBLK = NPAD // 128

E_TOTAL = 320000
CK = 112
NCH = 93
EP = NW * NCH * CK
ROWS_PER_TILE = NPAD // NS
ACC_R = 10112
ACC_PT = ACC_R // NS

PG = 640
PBR = PG // NS
BK = 64
BPT = NPAD // NW
NBCH = BPT // BK


def _sc_mesh():
    return plsc.VectorSubcoreMesh(
        core_axis_name="c", subcore_axis_name="s",
        num_cores=NC, num_subcores=NS)


def _fill_2d(ref, rows, cols, value):
    v = jnp.full((16,), value, jnp.float32)
    cpr = cols // 16

    def st(t, carry):
        ref[t // cpr, pl.ds((t % cpr) * 16, 16)] = v
        return carry

    lax.fori_loop(0, rows * cpr, st, 0)


def _unpack_chunk(pidx_c, sidx_c, didx_c, r):
    def st(t, carry):
        v = pidx_c[r, pl.ds(t * 16, 16)]
        sidx_c[r, pl.ds(t * 16, 16)] = lax.bitwise_and(v, 16383)
        didx_c[r, pl.ds(t * 16, 16)] = lax.shift_right_logical(v, 14)
        return carry

    lax.fori_loop(0, CK // 16, st, 0)


def _sc_degree(epack):
    def body(e_hbm, out_hbm, idx_v, ones_v, zero_v, accum):
        c = lax.axis_index("c")
        s = lax.axis_index("s")
        wid = s * NC + c
        _fill_2d(ones_v, CK, 16, 1.0)
        _fill_2d(zero_v, CK, 16, 0.0)
        pltpu.sync_copy(e_hbm.at[wid], idx_v)
        cpr = CK // 16

        def st(t, carry):
            r = t // cpr
            o = (t % cpr) * 16
            idx_v[r, pl.ds(o, 16)] = lax.shift_right_logical(
                idx_v[r, pl.ds(o, 16)], 14)
            return carry

        lax.fori_loop(0, NCH * cpr, st, 0)
        base = s * ROWS_PER_TILE
        for r in range(ROWS_PER_TILE // CK):
            pltpu.sync_copy(zero_v, accum.at[pl.ds(base + r * CK, CK)])
        zrem = ROWS_PER_TILE % CK
        if zrem:
            pltpu.sync_copy(
                zero_v.at[pl.ds(0, zrem)],
                accum.at[pl.ds(base + ROWS_PER_TILE - zrem, zrem)])
        plsc.subcore_barrier()

        def chunk(j, carry):
            pltpu.sync_copy(ones_v, accum.at[idx_v.at[j]], add=True)
            return carry

        lax.fori_loop(0, NCH, chunk, 0)
        plsc.subcore_barrier()
        pltpu.sync_copy(accum.at[pl.ds(base, ROWS_PER_TILE)],
                        out_hbm.at[c, pl.ds(base, ROWS_PER_TILE)])

    return pl.kernel(
        body,
        out_type=jax.ShapeDtypeStruct((NC, NPAD, 16), jnp.float32),
        mesh=_sc_mesh(),
        scratch_types=[
            pltpu.VMEM((NCH, CK), jnp.int32),
            pltpu.VMEM((CK, 16), jnp.float32),
            pltpu.VMEM((CK, 16), jnp.float32),
            pltpu.VMEM_SHARED((NPAD, 16), jnp.float32),
        ],
    )(epack)


def _sc_scatter(g, epack4):
    def body(g_hbm, e_hbm, out_hbm, pidx_c, sidx_c, didx_c, bufs, accum,
             gsem0, gsem1, gsem2, psem):
        gsems = (gsem0, gsem1, gsem2)
        c = lax.axis_index("c")
        s = lax.axis_index("s")
        wid = s * NC + c

        def zr(t, carry):
            bufs[0, t // 8, pl.ds((t % 8) * 16, 16)] = jnp.zeros(
                (16,), jnp.float32)
            return carry

        lax.fori_loop(0, CK * 8, zr, 0)
        base = s * ACC_PT
        for r in range(ACC_PT // CK):
            pltpu.sync_copy(bufs.at[0], accum.at[pl.ds(base + r * CK, CK)])
        rem = ACC_PT % CK
        pltpu.sync_copy(bufs.at[0, pl.ds(0, rem)],
                        accum.at[pl.ds(base + ACC_PT - rem, rem)])
        plsc.subcore_barrier()

        def load_packed(k, r):
            pltpu.async_copy(e_hbm.at[k, wid, 0], pidx_c.at[r], psem)

        def wait_packed(r):
            pltpu.make_async_copy(
                e_hbm.at[0, wid, 0], pidx_c.at[r], psem).wait()

        def issue_gather(r):
            pltpu.async_copy(g_hbm.at[sidx_c.at[r]], bufs.at[r], gsems[r])

        def wait_gather(r):
            pltpu.make_async_copy(
                g_hbm.at[sidx_c.at[r]], bufs.at[r], gsems[r]).wait()

        def one(k, carry):
            load_packed(k, 0)
            wait_packed(0)
            _unpack_chunk(pidx_c, sidx_c, didx_c, 0)
            issue_gather(0)
            wait_gather(0)
            pltpu.sync_copy(bufs.at[0], accum.at[didx_c.at[0]], add=True)
            return carry

        lax.fori_loop(0, NCH, one, 0)
        plsc.subcore_barrier()
        pltpu.sync_copy(accum.at[pl.ds(base, ACC_PT)],
                        out_hbm.at[c, pl.ds(base, ACC_PT)])

    return pl.kernel(
        body,
        out_type=jax.ShapeDtypeStruct((NC, NPAD, EMB), jnp.float32),
        mesh=_sc_mesh(),
        scratch_types=[
            pltpu.VMEM((8, CK), jnp.int32),
            pltpu.VMEM((8, CK), jnp.int32),
            pltpu.VMEM((8, CK), jnp.int32),
            pltpu.VMEM((3, CK, EMB), jnp.float32),
            pltpu.VMEM_SHARED((ACC_R, EMB), jnp.float32),
            pltpu.SemaphoreType.DMA,
            pltpu.SemaphoreType.DMA,
            pltpu.SemaphoreType.DMA,
            pltpu.SemaphoreType.DMA,
        ],
    )(g, epack4)


def _sc_pool(h, batch3):
    def body(h_hbm, b_hbm, psum_hbm, cnt_hbm,
             bidx, rows_v, ones_v, zp_v, zc_v, paccum, caccum):
        c = lax.axis_index("c")
        s = lax.axis_index("s")
        wid = s * NC + c
        _fill_2d(ones_v, BK, 16, 1.0)
        _fill_2d(zp_v, PBR, EMB, 0.0)
        _fill_2d(zc_v, PBR, 16, 0.0)
        pltpu.sync_copy(b_hbm.at[wid], bidx)
        pltpu.sync_copy(zp_v, paccum.at[pl.ds(s * PBR, PBR)])
        pltpu.sync_copy(zc_v, caccum.at[pl.ds(s * PBR, PBR)])
        plsc.subcore_barrier()

        def chunk(t, carry):
            pltpu.sync_copy(h_hbm.at[pl.ds(wid * BPT + t * BK, BK)], rows_v)
            pltpu.sync_copy(rows_v, paccum.at[bidx.at[t]], add=True)
            pltpu.sync_copy(ones_v, caccum.at[bidx.at[t]], add=True)
            return carry

        lax.fori_loop(0, NBCH, chunk, 0)
        plsc.subcore_barrier()
        pltpu.sync_copy(paccum.at[pl.ds(s * PBR, PBR)],
                        psum_hbm.at[c, pl.ds(s * PBR, PBR)])
        pltpu.sync_copy(caccum.at[pl.ds(s * PBR, PBR)],
                        cnt_hbm.at[c, pl.ds(s * PBR, PBR)])

    return pl.kernel(
        body,
        out_type=(jax.ShapeDtypeStruct((NC, PG, EMB), jnp.float32),
                  jax.ShapeDtypeStruct((NC, PG, 16), jnp.float32)),
        mesh=_sc_mesh(),
        scratch_types=[
            pltpu.VMEM((NBCH, BK), jnp.int32),
            pltpu.VMEM((BK, EMB), jnp.float32),
            pltpu.VMEM((BK, 16), jnp.float32),
            pltpu.VMEM((PBR, EMB), jnp.float32),
            pltpu.VMEM((PBR, 16), jnp.float32),
            pltpu.VMEM_SHARED((PG, EMB), jnp.float32),
            pltpu.VMEM_SHARED((PG, 16), jnp.float32),
        ],
    )(h, batch3)


def _tc_dinv(degp):
    def body(d0_ref, d1_ref, o_ref):
        i = pl.program_id(0)
        deg = 1.0 + d0_ref[0, :, 0:1] + d1_ref[0, :, 0:1]
        dinv = lax.rsqrt(deg)
        rid = i * 128 + lax.broadcasted_iota(jnp.int32, (128, 1), 0)
        dinv = jnp.where(rid < N_NODES, dinv, 0.0)
        o_ref[...] = jnp.broadcast_to(dinv, (128, EMB))

    return pl.pallas_call(
        body,
        grid=(NBLK,),
        in_specs=[pl.BlockSpec((1, 128, 16), lambda i: (0, i, 0)),
                  pl.BlockSpec((1, 128, 16), lambda i: (1, i, 0))],
        out_specs=pl.BlockSpec((128, EMB), lambda i: (i, 0)),
        out_shape=jax.ShapeDtypeStruct((NPAD, EMB), jnp.float32),
    )(degp, degp)


def _tc_entry(x, w, bias, dinvb):
    def body(x_ref, w_ref, b_ref, dv_ref, o_ref):
        h = jnp.dot(x_ref[...], w_ref[...], preferred_element_type=jnp.float32)
        o_ref[...] = dv_ref[...] * (h + b_ref[...])

    return pl.pallas_call(
        body,
        grid=(NBLK,),
        in_specs=[pl.BlockSpec((128, EMB), lambda i: (i, 0)),
                  pl.BlockSpec((EMB, EMB), lambda i: (0, 0)),
                  pl.BlockSpec((1, EMB), lambda i: (0, 0)),
                  pl.BlockSpec((128, EMB), lambda i: (i, 0))],
        out_specs=pl.BlockSpec((128, EMB), lambda i: (i, 0)),
        out_shape=jax.ShapeDtypeStruct((NPAD, EMB), jnp.float32),
    )(x, w, bias, dinvb)


def _tc_agg_stats(sp, g, dinvb):
    def body(s0_ref, s1_ref, g_ref, dv_ref, agg_ref, sum_ref, ssq_ref):
        i = pl.program_id(0)
        agg = dv_ref[...] * (s0_ref[0] + s1_ref[0] + g_ref[...])
        rid = i * 128 + lax.broadcasted_iota(jnp.int32, (128, 1), 0)
        agg = jnp.where(rid < N_NODES, agg, 0.0)
        agg_ref[...] = agg

        @pl.when(i == 0)
        def _():
            sum_ref[...] = jnp.zeros_like(sum_ref)
            ssq_ref[...] = jnp.zeros_like(ssq_ref)

        sum_ref[...] += jnp.broadcast_to(
            jnp.sum(agg, axis=0, keepdims=True), (8, EMB))
        ssq_ref[...] += jnp.broadcast_to(
            jnp.sum(agg * agg, axis=0, keepdims=True), (8, EMB))

    return pl.pallas_call(
        body,
        grid=(NBLK,),
        in_specs=[pl.BlockSpec((1, 128, EMB), lambda i: (0, i, 0)),
                  pl.BlockSpec((1, 128, EMB), lambda i: (1, i, 0)),
                  pl.BlockSpec((128, EMB), lambda i: (i, 0)),
                  pl.BlockSpec((128, EMB), lambda i: (i, 0))],
        out_specs=(pl.BlockSpec((128, EMB), lambda i: (i, 0)),
                   pl.BlockSpec((8, EMB), lambda i: (0, 0)),
                   pl.BlockSpec((8, EMB), lambda i: (0, 0))),
        out_shape=(jax.ShapeDtypeStruct((NPAD, EMB), jnp.float32),
                   jax.ShapeDtypeStruct((8, EMB), jnp.float32),
                   jax.ShapeDtypeStruct((8, EMB), jnp.float32)),
    )(sp, sp, g, dinvb)


def _tc_mid(agg, ssum, ssq, gam, bet, w, bias, dinvb):
    def body(agg_ref, sum_ref, ssq_ref, gam_ref, bet_ref, w_ref, b_ref,
             dv_ref, o_ref):
        inv_n = 1.0 / N_NODES
        mean = sum_ref[0:1, :] * inv_n
        var = ssq_ref[0:1, :] * inv_n - mean * mean
        a = gam_ref[...] * lax.rsqrt(var + BN_EPS)
        csh = bet_ref[...] - mean * a
        u = jnp.maximum(agg_ref[...] * a + csh, 0.0)
        h = jnp.dot(u, w_ref[...], preferred_element_type=jnp.float32)
        o_ref[...] = dv_ref[...] * (h + b_ref[...])

    return pl.pallas_call(
        body,
        grid=(NBLK,),
        in_specs=[pl.BlockSpec((128, EMB), lambda i: (i, 0)),
                  pl.BlockSpec((8, EMB), lambda i: (0, 0)),
                  pl.BlockSpec((8, EMB), lambda i: (0, 0)),
                  pl.BlockSpec((1, EMB), lambda i: (0, 0)),
                  pl.BlockSpec((1, EMB), lambda i: (0, 0)),
                  pl.BlockSpec((EMB, EMB), lambda i: (0, 0)),
                  pl.BlockSpec((1, EMB), lambda i: (0, 0)),
                  pl.BlockSpec((128, EMB), lambda i: (i, 0))],
        out_specs=pl.BlockSpec((128, EMB), lambda i: (i, 0)),
        out_shape=jax.ShapeDtypeStruct((NPAD, EMB), jnp.float32),
    )(agg, ssum, ssq, gam, bet, w, bias, dinvb)


def _tc_last(agg, ssum, ssq, gam, bet):
    def body(agg_ref, sum_ref, ssq_ref, gam_ref, bet_ref, o_ref):
        inv_n = 1.0 / N_NODES
        mean = sum_ref[0:1, :] * inv_n
        var = ssq_ref[0:1, :] * inv_n - mean * mean
        a = gam_ref[...] * lax.rsqrt(var + BN_EPS)
        csh = bet_ref[...] - mean * a
        o_ref[...] = agg_ref[...] * a + csh

    return pl.pallas_call(
        body,
        grid=(NBLK,),
        in_specs=[pl.BlockSpec((128, EMB), lambda i: (i, 0)),
                  pl.BlockSpec((8, EMB), lambda i: (0, 0)),
                  pl.BlockSpec((8, EMB), lambda i: (0, 0)),
                  pl.BlockSpec((1, EMB), lambda i: (0, 0)),
                  pl.BlockSpec((1, EMB), lambda i: (0, 0))],
        out_specs=pl.BlockSpec((128, EMB), lambda i: (i, 0)),
        out_shape=jax.ShapeDtypeStruct((NPAD, EMB), jnp.float32),
    )(agg, ssum, ssq, gam, bet)


def _tc_pool_div(psum, cnt):
    def body(p0_ref, p1_ref, c0_ref, c1_ref, o_ref):
        cc = c0_ref[0, :, 0:1] + c1_ref[0, :, 0:1]
        o_ref[...] = (p0_ref[0] + p1_ref[0]) / jnp.maximum(cc, 1.0)

    return pl.pallas_call(
        body,
        grid=(1,),
        in_specs=[pl.BlockSpec((1, NUM_GRAPHS, EMB), lambda i: (0, 0, 0)),
                  pl.BlockSpec((1, NUM_GRAPHS, EMB), lambda i: (1, 0, 0)),
                  pl.BlockSpec((1, NUM_GRAPHS, 16), lambda i: (0, 0, 0)),
                  pl.BlockSpec((1, NUM_GRAPHS, 16), lambda i: (1, 0, 0))],
        out_specs=pl.BlockSpec((NUM_GRAPHS, EMB), lambda i: (0, 0)),
        out_shape=jax.ShapeDtypeStruct((NUM_GRAPHS, EMB), jnp.float32),
    )(psum, psum, cnt, cnt)


def kernel(x, edge_index, edge_attr, batch, W, b, gamma, beta):
    del edge_attr
    f32 = jnp.float32

    src = edge_index[0].astype(jnp.int32)
    dst = edge_index[1].astype(jnp.int32)
    packed = src + dst * 16384
    pad_e = jnp.full((EP - E_TOTAL,), PAD_ROW + PAD_ROW * 16384,
                     dtype=jnp.int32)
    epack = jnp.concatenate([packed, pad_e]).reshape(NW, NCH, CK)
    epack4 = jnp.transpose(epack, (1, 0, 2)).reshape(NCH, NW, 1, CK)
    batch3 = jnp.concatenate(
        [batch.astype(jnp.int32),
         jnp.full((NPAD - N_NODES,), NUM_GRAPHS, dtype=jnp.int32)]
    ).reshape(NW, NBCH, BK)
    x_pad = jnp.concatenate(
        [x.astype(f32), jnp.zeros((NPAD - N_NODES, EMB), f32)], axis=0)

    degp = _sc_degree(epack)
    dinvb = _tc_dinv(degp)
    g = _tc_entry(x_pad, W[0], b[0].reshape(1, EMB), dinvb)

    h_final = None
    for l in range(NUM_LAYER):
        sp = _sc_scatter(g, epack4)
        agg, ssum, ssq = _tc_agg_stats(sp, g, dinvb)
        gam = gamma[l].reshape(1, EMB)
        bet = beta[l].reshape(1, EMB)
        if l < NUM_LAYER - 1:
            g = _tc_mid(agg, ssum, ssq, gam, bet,
                        W[l + 1], b[l + 1].reshape(1, EMB), dinvb)
        else:
            h_final = _tc_last(agg, ssum, ssq, gam, bet)

    psum, cnt = _sc_pool(h_final, batch3)
    return _tc_pool_div(psum, cnt)


# speedup vs baseline: 1.1149x; 1.1149x over previous
import jax
import jax.numpy as jnp
from jax import lax
from jax.experimental import pallas as pl
from jax.experimental.pallas import tpu as pltpu
from jax.experimental.pallas import tpu_sc as plsc

N_NODES = 10000
EMB = 128
NUM_LAYER = 5
NUM_GRAPHS = 512
BN_EPS = 1e-5

NC = 2
NS = 16
NW = NC * NS

NPAD = 10240
PAD_ROW = N_NODES
NBLK = NPAD // 128

E_TOTAL = 320000
CK = 112
NCH = 93
EP = NW * NCH * CK
ROWS_PER_TILE = NPAD // NS
ACC_R = 10112
ACC_PT = ACC_R // NS

PG = 640
PBR = PG // NS
BK = 64
BPT = NPAD // NW
NBCH = BPT // BK


def _sc_mesh():
    return plsc.VectorSubcoreMesh(
        core_axis_name="c", subcore_axis_name="s",
        num_cores=NC, num_subcores=NS)


def _fill_2d(ref, rows, cols, value):
    v = jnp.full((16,), value, jnp.float32)
    cpr = cols // 16

    def st(t, carry):
        ref[t // cpr, pl.ds((t % cpr) * 16, 16)] = v
        return carry

    lax.fori_loop(0, rows * cpr, st, 0)


def _unpack_chunk(pidx_c, sidx_c, didx_c, r):
    def st(t, carry):
        v = pidx_c[r, pl.ds(t * 16, 16)]
        sidx_c[r, pl.ds(t * 16, 16)] = lax.bitwise_and(v, 16383)
        didx_c[r, pl.ds(t * 16, 16)] = lax.shift_right_logical(v, 14)
        return carry

    lax.fori_loop(0, CK // 16, st, 0)


def _sc_degree(epack):
    def body(e_hbm, out_hbm, idx_v, ones_v, zero_v, accum):
        c = lax.axis_index("c")
        s = lax.axis_index("s")
        wid = s * NC + c
        _fill_2d(ones_v, CK, 16, 1.0)
        _fill_2d(zero_v, CK, 16, 0.0)
        pltpu.sync_copy(e_hbm.at[wid], idx_v)
        cpr = CK // 16

        def st(t, carry):
            r = t // cpr
            o = (t % cpr) * 16
            idx_v[r, pl.ds(o, 16)] = lax.shift_right_logical(
                idx_v[r, pl.ds(o, 16)], 14)
            return carry

        lax.fori_loop(0, NCH * cpr, st, 0)
        base = s * ROWS_PER_TILE
        for r in range(ROWS_PER_TILE // CK):
            pltpu.sync_copy(zero_v, accum.at[pl.ds(base + r * CK, CK)])
        zrem = ROWS_PER_TILE % CK
        if zrem:
            pltpu.sync_copy(
                zero_v.at[pl.ds(0, zrem)],
                accum.at[pl.ds(base + ROWS_PER_TILE - zrem, zrem)])
        plsc.subcore_barrier()

        def chunk(j, carry):
            pltpu.sync_copy(ones_v, accum.at[idx_v.at[j]], add=True)
            return carry

        lax.fori_loop(0, NCH, chunk, 0)
        plsc.subcore_barrier()
        pltpu.sync_copy(accum.at[pl.ds(base, ROWS_PER_TILE)],
                        out_hbm.at[c, pl.ds(base, ROWS_PER_TILE)])

    return pl.kernel(
        body,
        out_type=jax.ShapeDtypeStruct((NC, NPAD, 16), jnp.float32),
        mesh=_sc_mesh(),
        scratch_types=[
            pltpu.VMEM((NCH, CK), jnp.int32),
            pltpu.VMEM((CK, 16), jnp.float32),
            pltpu.VMEM((CK, 16), jnp.float32),
            pltpu.VMEM_SHARED((NPAD, 16), jnp.float32),
        ],
    )(epack)


def _sc_scatter(g, epack4):
    def body(g_hbm, e_hbm, out_hbm, pidx_c, sidx_c, didx_c, bufs, accum,
             gsem0, gsem1, gsem2, psem):
        gsems = (gsem0, gsem1, gsem2)
        c = lax.axis_index("c")
        s = lax.axis_index("s")
        wid = s * NC + c

        def zr(t, carry):
            bufs[0, t // 8, pl.ds((t % 8) * 16, 16)] = jnp.zeros(
                (16,), jnp.float32)
            return carry

        lax.fori_loop(0, CK * 8, zr, 0)
        base = s * ACC_PT
        for r in range(ACC_PT // CK):
            pltpu.sync_copy(bufs.at[0], accum.at[pl.ds(base + r * CK, CK)])
        rem = ACC_PT % CK
        pltpu.sync_copy(bufs.at[0, pl.ds(0, rem)],
                        accum.at[pl.ds(base + ACC_PT - rem, rem)])
        plsc.subcore_barrier()

        def load_packed(k, r):
            pltpu.async_copy(e_hbm.at[k, wid, 0], pidx_c.at[r], psem)

        def wait_packed(r):
            pltpu.make_async_copy(
                e_hbm.at[0, wid, 0], pidx_c.at[r], psem).wait()

        def issue_gather(r):
            pltpu.async_copy(g_hbm.at[sidx_c.at[r]], bufs.at[r], gsems[r])

        def wait_gather(r):
            pltpu.make_async_copy(
                g_hbm.at[sidx_c.at[r]], bufs.at[r], gsems[r]).wait()

        load_packed(0, 0)
        wait_packed(0)
        _unpack_chunk(pidx_c, sidx_c, didx_c, 0)
        issue_gather(0)
        load_packed(1, 1)
        wait_packed(1)
        _unpack_chunk(pidx_c, sidx_c, didx_c, 1)
        issue_gather(1)
        load_packed(2, 2)

        def tri(j, carry):
            for u in range(3):
                e = 3 * j + u
                r0 = u
                r2 = (u + 2) % 3

                @pl.when(e + 2 < NCH)
                def _():
                    wait_packed(r2)
                    _unpack_chunk(pidx_c, sidx_c, didx_c, r2)
                    issue_gather(r2)

                wait_gather(r0)
                pltpu.sync_copy(bufs.at[r0], accum.at[didx_c.at[r0]],
                                add=True)

                @pl.when(e + 3 < NCH)
                def _():
                    load_packed(e + 3, r0)
            return carry

        lax.fori_loop(0, NCH // 3, tri, 0)
        plsc.subcore_barrier()
        pltpu.sync_copy(accum.at[pl.ds(base, ACC_PT)],
                        out_hbm.at[c, pl.ds(base, ACC_PT)])

    return pl.kernel(
        body,
        out_type=jax.ShapeDtypeStruct((NC, NPAD, EMB), jnp.float32),
        mesh=_sc_mesh(),
        scratch_types=[
            pltpu.VMEM((8, CK), jnp.int32),
            pltpu.VMEM((8, CK), jnp.int32),
            pltpu.VMEM((8, CK), jnp.int32),
            pltpu.VMEM((3, CK, EMB), jnp.float32),
            pltpu.VMEM_SHARED((ACC_R, EMB), jnp.float32),
            pltpu.SemaphoreType.DMA,
            pltpu.SemaphoreType.DMA,
            pltpu.SemaphoreType.DMA,
            pltpu.SemaphoreType.DMA,
        ],
    )(g, epack4)


def _sc_pool(h, batch3):
    def body(h_hbm, b_hbm, psum_hbm, cnt_hbm,
             bidx, rows_v, ones_v, zp_v, zc_v, paccum, caccum):
        c = lax.axis_index("c")
        s = lax.axis_index("s")
        wid = s * NC + c
        _fill_2d(ones_v, BK, 16, 1.0)
        _fill_2d(zp_v, PBR, EMB, 0.0)
        _fill_2d(zc_v, PBR, 16, 0.0)
        pltpu.sync_copy(b_hbm.at[wid], bidx)
        pltpu.sync_copy(zp_v, paccum.at[pl.ds(s * PBR, PBR)])
        pltpu.sync_copy(zc_v, caccum.at[pl.ds(s * PBR, PBR)])
        plsc.subcore_barrier()

        def chunk(t, carry):
            pltpu.sync_copy(h_hbm.at[pl.ds(wid * BPT + t * BK, BK)], rows_v)
            pltpu.sync_copy(rows_v, paccum.at[bidx.at[t]], add=True)
            pltpu.sync_copy(ones_v, caccum.at[bidx.at[t]], add=True)
            return carry

        lax.fori_loop(0, NBCH, chunk, 0)
        plsc.subcore_barrier()
        pltpu.sync_copy(paccum.at[pl.ds(s * PBR, PBR)],
                        psum_hbm.at[c, pl.ds(s * PBR, PBR)])
        pltpu.sync_copy(caccum.at[pl.ds(s * PBR, PBR)],
                        cnt_hbm.at[c, pl.ds(s * PBR, PBR)])

    return pl.kernel(
        body,
        out_type=(jax.ShapeDtypeStruct((NC, PG, EMB), jnp.float32),
                  jax.ShapeDtypeStruct((NC, PG, 16), jnp.float32)),
        mesh=_sc_mesh(),
        scratch_types=[
            pltpu.VMEM((NBCH, BK), jnp.int32),
            pltpu.VMEM((BK, EMB), jnp.float32),
            pltpu.VMEM((BK, 16), jnp.float32),
            pltpu.VMEM((PBR, EMB), jnp.float32),
            pltpu.VMEM((PBR, 16), jnp.float32),
            pltpu.VMEM_SHARED((PG, EMB), jnp.float32),
            pltpu.VMEM_SHARED((PG, 16), jnp.float32),
        ],
    )(h, batch3)


def _tc_dinv(degp):
    def body(d0_ref, d1_ref, o_ref):
        i = pl.program_id(0)
        deg = 1.0 + d0_ref[0, :, 0:1] + d1_ref[0, :, 0:1]
        dinv = lax.rsqrt(deg)
        rid = i * 128 + lax.broadcasted_iota(jnp.int32, (128, 1), 0)
        dinv = jnp.where(rid < N_NODES, dinv, 0.0)
        o_ref[...] = jnp.broadcast_to(dinv, (128, EMB))

    return pl.pallas_call(
        body,
        grid=(NBLK,),
        in_specs=[pl.BlockSpec((1, 128, 16), lambda i: (0, i, 0)),
                  pl.BlockSpec((1, 128, 16), lambda i: (1, i, 0))],
        out_specs=pl.BlockSpec((128, EMB), lambda i: (i, 0)),
        out_shape=jax.ShapeDtypeStruct((NPAD, EMB), jnp.float32),
    )(degp, degp)


def _tc_entry(x, w, bias, dinvb):
    def body(x_ref, w_ref, b_ref, dv_ref, o_ref):
        h = jnp.dot(x_ref[...], w_ref[...], preferred_element_type=jnp.float32)
        o_ref[...] = dv_ref[...] * (h + b_ref[...])

    return pl.pallas_call(
        body,
        grid=(NBLK,),
        in_specs=[pl.BlockSpec((128, EMB), lambda i: (i, 0)),
                  pl.BlockSpec((EMB, EMB), lambda i: (0, 0)),
                  pl.BlockSpec((1, EMB), lambda i: (0, 0)),
                  pl.BlockSpec((128, EMB), lambda i: (i, 0))],
        out_specs=pl.BlockSpec((128, EMB), lambda i: (i, 0)),
        out_shape=jax.ShapeDtypeStruct((NPAD, EMB), jnp.float32),
    )(x, w, bias, dinvb)


def _tc_agg_stats(sp, g, dinvb):
    def body(s0_ref, s1_ref, g_ref, dv_ref, agg_ref, sum_ref, ssq_ref):
        i = pl.program_id(0)
        agg = dv_ref[...] * (s0_ref[0] + s1_ref[0] + g_ref[...])
        rid = i * 128 + lax.broadcasted_iota(jnp.int32, (128, 1), 0)
        agg = jnp.where(rid < N_NODES, agg, 0.0)
        agg_ref[...] = agg

        @pl.when(i == 0)
        def _():
            sum_ref[...] = jnp.zeros_like(sum_ref)
            ssq_ref[...] = jnp.zeros_like(ssq_ref)

        sum_ref[...] += jnp.broadcast_to(
            jnp.sum(agg, axis=0, keepdims=True), (8, EMB))
        ssq_ref[...] += jnp.broadcast_to(
            jnp.sum(agg * agg, axis=0, keepdims=True), (8, EMB))

    return pl.pallas_call(
        body,
        grid=(NBLK,),
        in_specs=[pl.BlockSpec((1, 128, EMB), lambda i: (0, i, 0)),
                  pl.BlockSpec((1, 128, EMB), lambda i: (1, i, 0)),
                  pl.BlockSpec((128, EMB), lambda i: (i, 0)),
                  pl.BlockSpec((128, EMB), lambda i: (i, 0))],
        out_specs=(pl.BlockSpec((128, EMB), lambda i: (i, 0)),
                   pl.BlockSpec((8, EMB), lambda i: (0, 0)),
                   pl.BlockSpec((8, EMB), lambda i: (0, 0))),
        out_shape=(jax.ShapeDtypeStruct((NPAD, EMB), jnp.float32),
                   jax.ShapeDtypeStruct((8, EMB), jnp.float32),
                   jax.ShapeDtypeStruct((8, EMB), jnp.float32)),
    )(sp, sp, g, dinvb)


def _tc_mid(agg, ssum, ssq, gam, bet, w, bias, dinvb):
    def body(agg_ref, sum_ref, ssq_ref, gam_ref, bet_ref, w_ref, b_ref,
             dv_ref, o_ref):
        inv_n = 1.0 / N_NODES
        mean = sum_ref[0:1, :] * inv_n
        var = ssq_ref[0:1, :] * inv_n - mean * mean
        a = gam_ref[...] * lax.rsqrt(var + BN_EPS)
        csh = bet_ref[...] - mean * a
        u = jnp.maximum(agg_ref[...] * a + csh, 0.0)
        h = jnp.dot(u, w_ref[...], preferred_element_type=jnp.float32)
        o_ref[...] = dv_ref[...] * (h + b_ref[...])

    return pl.pallas_call(
        body,
        grid=(NBLK,),
        in_specs=[pl.BlockSpec((128, EMB), lambda i: (i, 0)),
                  pl.BlockSpec((8, EMB), lambda i: (0, 0)),
                  pl.BlockSpec((8, EMB), lambda i: (0, 0)),
                  pl.BlockSpec((1, EMB), lambda i: (0, 0)),
                  pl.BlockSpec((1, EMB), lambda i: (0, 0)),
                  pl.BlockSpec((EMB, EMB), lambda i: (0, 0)),
                  pl.BlockSpec((1, EMB), lambda i: (0, 0)),
                  pl.BlockSpec((128, EMB), lambda i: (i, 0))],
        out_specs=pl.BlockSpec((128, EMB), lambda i: (i, 0)),
        out_shape=jax.ShapeDtypeStruct((NPAD, EMB), jnp.float32),
    )(agg, ssum, ssq, gam, bet, w, bias, dinvb)


def _tc_last(agg, ssum, ssq, gam, bet):
    def body(agg_ref, sum_ref, ssq_ref, gam_ref, bet_ref, o_ref):
        inv_n = 1.0 / N_NODES
        mean = sum_ref[0:1, :] * inv_n
        var = ssq_ref[0:1, :] * inv_n - mean * mean
        a = gam_ref[...] * lax.rsqrt(var + BN_EPS)
        csh = bet_ref[...] - mean * a
        o_ref[...] = agg_ref[...] * a + csh

    return pl.pallas_call(
        body,
        grid=(NBLK,),
        in_specs=[pl.BlockSpec((128, EMB), lambda i: (i, 0)),
                  pl.BlockSpec((8, EMB), lambda i: (0, 0)),
                  pl.BlockSpec((8, EMB), lambda i: (0, 0)),
                  pl.BlockSpec((1, EMB), lambda i: (0, 0)),
                  pl.BlockSpec((1, EMB), lambda i: (0, 0))],
        out_specs=pl.BlockSpec((128, EMB), lambda i: (i, 0)),
        out_shape=jax.ShapeDtypeStruct((NPAD, EMB), jnp.float32),
    )(agg, ssum, ssq, gam, bet)


def _tc_pool_div(psum, cnt):
    def body(p0_ref, p1_ref, c0_ref, c1_ref, o_ref):
        cc = c0_ref[0, :, 0:1] + c1_ref[0, :, 0:1]
        o_ref[...] = (p0_ref[0] + p1_ref[0]) / jnp.maximum(cc, 1.0)

    return pl.pallas_call(
        body,
        grid=(1,),
        in_specs=[pl.BlockSpec((1, NUM_GRAPHS, EMB), lambda i: (0, 0, 0)),
                  pl.BlockSpec((1, NUM_GRAPHS, EMB), lambda i: (1, 0, 0)),
                  pl.BlockSpec((1, NUM_GRAPHS, 16), lambda i: (0, 0, 0)),
                  pl.BlockSpec((1, NUM_GRAPHS, 16), lambda i: (1, 0, 0))],
        out_specs=pl.BlockSpec((NUM_GRAPHS, EMB), lambda i: (0, 0)),
        out_shape=jax.ShapeDtypeStruct((NUM_GRAPHS, EMB), jnp.float32),
    )(psum, psum, cnt, cnt)


def kernel(x, edge_index, edge_attr, batch, W, b, gamma, beta):
    del edge_attr
    f32 = jnp.float32

    src = edge_index[0].astype(jnp.int32)
    dst = edge_index[1].astype(jnp.int32)
    packed = src + dst * 16384
    pad_e = jnp.full((EP - E_TOTAL,), PAD_ROW + PAD_ROW * 16384,
                     dtype=jnp.int32)
    epack = jnp.concatenate([packed, pad_e]).reshape(NW, NCH, CK)
    epack4 = jnp.transpose(epack, (1, 0, 2)).reshape(NCH, NW, 1, CK)
    batch3 = jnp.concatenate(
        [batch.astype(jnp.int32),
         jnp.full((NPAD - N_NODES,), NUM_GRAPHS, dtype=jnp.int32)]
    ).reshape(NW, NBCH, BK)
    x_pad = jnp.concatenate(
        [x.astype(f32), jnp.zeros((NPAD - N_NODES, EMB), f32)], axis=0)

    degp = _sc_degree(epack)
    dinvb = _tc_dinv(degp)
    g = _tc_entry(x_pad, W[0], b[0].reshape(1, EMB), dinvb)

    h_final = None
    for l in range(NUM_LAYER):
        sp = _sc_scatter(g, epack4)
        agg, ssum, ssq = _tc_agg_stats(sp, g, dinvb)
        gam = gamma[l].reshape(1, EMB)
        bet = beta[l].reshape(1, EMB)
        if l < NUM_LAYER - 1:
            g = _tc_mid(agg, ssum, ssq, gam, bet,
                        W[l + 1], b[l + 1].reshape(1, EMB), dinvb)
        else:
            h_final = _tc_last(agg, ssum, ssq, gam, bet)

    psum, cnt = _sc_pool(h_final, batch3)
    return _tc_pool_div(psum, cnt)


# speedup vs baseline: 2.0725x; 1.8589x over previous
import jax
import jax.numpy as jnp
from jax import lax
from jax.experimental import pallas as pl
from jax.experimental.pallas import tpu as pltpu
from jax.experimental.pallas import tpu_sc as plsc

N_NODES = 10000
EMB = 128
NUM_LAYER = 5
NUM_GRAPHS = 512
BN_EPS = 1e-5

NC = 2
NS = 16
NW = NC * NS

NPAD = 10240
PAD_ROW = N_NODES
NBLK = NPAD // 128

E_TOTAL = 320000
CK = 112
FAST_C = 1
NCHF = 138
NCHS = 42
NCHM = NCHF
E_FAST = NS * NCHF * CK
E_SLOW = E_TOTAL - E_FAST
ROWS_PER_TILE = NPAD // NS
ACC_R = 10112
ACC_PT = ACC_R // NS

PG = 640
PBR = PG // NS
BK = 64
BPT = NPAD // NW
NBCH = BPT // BK


def _sc_mesh():
    return plsc.VectorSubcoreMesh(
        core_axis_name="c", subcore_axis_name="s",
        num_cores=NC, num_subcores=NS)


def _fill_2d(ref, rows, cols, value):
    v = jnp.full((16,), value, jnp.float32)
    cpr = cols // 16

    def st(t, carry):
        ref[t // cpr, pl.ds((t % cpr) * 16, 16)] = v
        return carry

    lax.fori_loop(0, rows * cpr, st, 0)


def _unpack_chunk(packed, sidx_c, didx_c, r, k):
    def st(t, carry):
        v = packed[k, pl.ds(t * 16, 16)]
        sidx_c[r, pl.ds(t * 16, 16)] = lax.bitwise_and(v, 16383)
        didx_c[r, pl.ds(t * 16, 16)] = lax.shift_right_logical(v, 14)
        return carry

    lax.fori_loop(0, CK // 16, st, 0)


def _sc_degree(epack):
    def body(e_hbm, out_hbm, idx_v, ones_v, zero_v, accum):
        c = lax.axis_index("c")
        s = lax.axis_index("s")
        wid = s * NC + c
        _fill_2d(ones_v, CK, 16, 1.0)
        _fill_2d(zero_v, CK, 16, 0.0)
        pltpu.sync_copy(e_hbm.at[wid], idx_v)
        cpr = CK // 16

        def st(t, carry):
            r = t // cpr
            o = (t % cpr) * 16
            idx_v[r, pl.ds(o, 16)] = lax.shift_right_logical(
                idx_v[r, pl.ds(o, 16)], 14)
            return carry

        lax.fori_loop(0, NCHM * cpr, st, 0)
        base = s * ROWS_PER_TILE
        for r in range(ROWS_PER_TILE // CK):
            pltpu.sync_copy(zero_v, accum.at[pl.ds(base + r * CK, CK)])
        zrem = ROWS_PER_TILE % CK
        if zrem:
            pltpu.sync_copy(
                zero_v.at[pl.ds(0, zrem)],
                accum.at[pl.ds(base + ROWS_PER_TILE - zrem, zrem)])
        plsc.subcore_barrier()

        def chunk(j, carry):
            pltpu.sync_copy(ones_v, accum.at[idx_v.at[j]], add=True)
            return carry

        lax.fori_loop(0, NCHM, chunk, 0)
        plsc.subcore_barrier()
        pltpu.sync_copy(accum.at[pl.ds(base, ROWS_PER_TILE)],
                        out_hbm.at[c, pl.ds(base, ROWS_PER_TILE)])

    return pl.kernel(
        body,
        out_type=jax.ShapeDtypeStruct((NC, NPAD, 16), jnp.float32),
        mesh=_sc_mesh(),
        scratch_types=[
            pltpu.VMEM((NCHM, CK), jnp.int32),
            pltpu.VMEM((CK, 16), jnp.float32),
            pltpu.VMEM((CK, 16), jnp.float32),
            pltpu.VMEM_SHARED((NPAD, 16), jnp.float32),
        ],
    )(epack)


def _sc_scatter(g, epack):
    def body(g_hbm, e_hbm, out_hbm, packed, sidx_c, didx_c, bufs, accum,
             sem):
        c = lax.axis_index("c")
        s = lax.axis_index("s")
        wid = s * NC + c
        pltpu.sync_copy(e_hbm.at[wid], packed)

        def zr(t, carry):
            bufs[0, t // 8, pl.ds((t % 8) * 16, 16)] = jnp.zeros(
                (16,), jnp.float32)
            return carry

        lax.fori_loop(0, CK * 8, zr, 0)
        base = s * ACC_PT
        for r in range(ACC_PT // CK):
            pltpu.sync_copy(bufs.at[0], accum.at[pl.ds(base + r * CK, CK)])
        rem = ACC_PT % CK
        pltpu.sync_copy(bufs.at[0, pl.ds(0, rem)],
                        accum.at[pl.ds(base + ACC_PT - rem, rem)])
        plsc.subcore_barrier()

        def issue_gather(r):
            pltpu.async_copy(g_hbm.at[sidx_c.at[r]], bufs.at[r], sem)

        def wait_gather(r):
            pltpu.make_async_copy(
                g_hbm.at[sidx_c.at[r]], bufs.at[r], sem).wait()

        _unpack_chunk(packed, sidx_c, didx_c, 0, 0)
        issue_gather(0)

        def make_pair(n_half):
            def pair(j, carry):
                e = 2 * j
                _unpack_chunk(packed, sidx_c, didx_c, 1, e + 1)
                wait_gather(0)
                issue_gather(1)
                pltpu.sync_copy(bufs.at[0], accum.at[didx_c.at[0]],
                                add=True)
                _unpack_chunk(packed, sidx_c, didx_c, 0,
                              jnp.minimum(e + 2, NCHM - 1))
                wait_gather(1)

                @pl.when(j < n_half - 1)
                def _():
                    issue_gather(0)

                pltpu.sync_copy(bufs.at[1], accum.at[didx_c.at[1]],
                                add=True)
                return carry

            return pair

        @pl.when(c == FAST_C)
        def _():
            lax.fori_loop(0, NCHF // 2, make_pair(NCHF // 2), 0)

        @pl.when(c != FAST_C)
        def _():
            lax.fori_loop(0, NCHS // 2, make_pair(NCHS // 2), 0)

        plsc.subcore_barrier()
        pltpu.sync_copy(accum.at[pl.ds(base, ACC_PT)],
                        out_hbm.at[c, pl.ds(base, ACC_PT)])

    return pl.kernel(
        body,
        out_type=jax.ShapeDtypeStruct((NC, NPAD, EMB), jnp.float32),
        mesh=_sc_mesh(),
        scratch_types=[
            pltpu.VMEM((NCHM, CK), jnp.int32),
            pltpu.VMEM((8, CK), jnp.int32),
            pltpu.VMEM((8, CK), jnp.int32),
            pltpu.VMEM((2, CK, EMB), jnp.float32),
            pltpu.VMEM_SHARED((ACC_R, EMB), jnp.float32),
            pltpu.SemaphoreType.DMA,
        ],
    )(g, epack)


def _sc_pool(h, batch3):
    def body(h_hbm, b_hbm, psum_hbm, cnt_hbm,
             bidx, rows_v, ones_v, zp_v, zc_v, paccum, caccum):
        c = lax.axis_index("c")
        s = lax.axis_index("s")
        wid = s * NC + c
        _fill_2d(ones_v, BK, 16, 1.0)
        _fill_2d(zp_v, PBR, EMB, 0.0)
        _fill_2d(zc_v, PBR, 16, 0.0)
        pltpu.sync_copy(b_hbm.at[wid], bidx)
        pltpu.sync_copy(zp_v, paccum.at[pl.ds(s * PBR, PBR)])
        pltpu.sync_copy(zc_v, caccum.at[pl.ds(s * PBR, PBR)])
        plsc.subcore_barrier()

        def chunk(t, carry):
            pltpu.sync_copy(h_hbm.at[pl.ds(wid * BPT + t * BK, BK)], rows_v)
            pltpu.sync_copy(rows_v, paccum.at[bidx.at[t]], add=True)
            pltpu.sync_copy(ones_v, caccum.at[bidx.at[t]], add=True)
            return carry

        lax.fori_loop(0, NBCH, chunk, 0)
        plsc.subcore_barrier()
        pltpu.sync_copy(paccum.at[pl.ds(s * PBR, PBR)],
                        psum_hbm.at[c, pl.ds(s * PBR, PBR)])
        pltpu.sync_copy(caccum.at[pl.ds(s * PBR, PBR)],
                        cnt_hbm.at[c, pl.ds(s * PBR, PBR)])

    return pl.kernel(
        body,
        out_type=(jax.ShapeDtypeStruct((NC, PG, EMB), jnp.float32),
                  jax.ShapeDtypeStruct((NC, PG, 16), jnp.float32)),
        mesh=_sc_mesh(),
        scratch_types=[
            pltpu.VMEM((NBCH, BK), jnp.int32),
            pltpu.VMEM((BK, EMB), jnp.float32),
            pltpu.VMEM((BK, 16), jnp.float32),
            pltpu.VMEM((PBR, EMB), jnp.float32),
            pltpu.VMEM((PBR, 16), jnp.float32),
            pltpu.VMEM_SHARED((PG, EMB), jnp.float32),
            pltpu.VMEM_SHARED((PG, 16), jnp.float32),
        ],
    )(h, batch3)


def _tc_dinv(degp):
    def body(d0_ref, d1_ref, o_ref):
        i = pl.program_id(0)
        deg = 1.0 + d0_ref[0, :, 0:1] + d1_ref[0, :, 0:1]
        dinv = lax.rsqrt(deg)
        rid = i * 128 + lax.broadcasted_iota(jnp.int32, (128, 1), 0)
        dinv = jnp.where(rid < N_NODES, dinv, 0.0)
        o_ref[...] = jnp.broadcast_to(dinv, (128, EMB))

    return pl.pallas_call(
        body,
        grid=(NBLK,),
        in_specs=[pl.BlockSpec((1, 128, 16), lambda i: (0, i, 0)),
                  pl.BlockSpec((1, 128, 16), lambda i: (1, i, 0))],
        out_specs=pl.BlockSpec((128, EMB), lambda i: (i, 0)),
        out_shape=jax.ShapeDtypeStruct((NPAD, EMB), jnp.float32),
    )(degp, degp)


def _tc_entry(x, w, bias, dinvb):
    def body(x_ref, w_ref, b_ref, dv_ref, o_ref):
        h = jnp.dot(x_ref[...], w_ref[...], preferred_element_type=jnp.float32)
        o_ref[...] = dv_ref[...] * (h + b_ref[...])

    return pl.pallas_call(
        body,
        grid=(NBLK,),
        in_specs=[pl.BlockSpec((128, EMB), lambda i: (i, 0)),
                  pl.BlockSpec((EMB, EMB), lambda i: (0, 0)),
                  pl.BlockSpec((1, EMB), lambda i: (0, 0)),
                  pl.BlockSpec((128, EMB), lambda i: (i, 0))],
        out_specs=pl.BlockSpec((128, EMB), lambda i: (i, 0)),
        out_shape=jax.ShapeDtypeStruct((NPAD, EMB), jnp.float32),
    )(x, w, bias, dinvb)


def _tc_agg_stats(sp, g, dinvb):
    def body(s0_ref, s1_ref, g_ref, dv_ref, agg_ref, sum_ref, ssq_ref):
        i = pl.program_id(0)
        agg = dv_ref[...] * (s0_ref[0] + s1_ref[0] + g_ref[...])
        rid = i * 128 + lax.broadcasted_iota(jnp.int32, (128, 1), 0)
        agg = jnp.where(rid < N_NODES, agg, 0.0)
        agg_ref[...] = agg

        @pl.when(i == 0)
        def _():
            sum_ref[...] = jnp.zeros_like(sum_ref)
            ssq_ref[...] = jnp.zeros_like(ssq_ref)

        sum_ref[...] += jnp.broadcast_to(
            jnp.sum(agg, axis=0, keepdims=True), (8, EMB))
        ssq_ref[...] += jnp.broadcast_to(
            jnp.sum(agg * agg, axis=0, keepdims=True), (8, EMB))

    return pl.pallas_call(
        body,
        grid=(NBLK,),
        in_specs=[pl.BlockSpec((1, 128, EMB), lambda i: (0, i, 0)),
                  pl.BlockSpec((1, 128, EMB), lambda i: (1, i, 0)),
                  pl.BlockSpec((128, EMB), lambda i: (i, 0)),
                  pl.BlockSpec((128, EMB), lambda i: (i, 0))],
        out_specs=(pl.BlockSpec((128, EMB), lambda i: (i, 0)),
                   pl.BlockSpec((8, EMB), lambda i: (0, 0)),
                   pl.BlockSpec((8, EMB), lambda i: (0, 0))),
        out_shape=(jax.ShapeDtypeStruct((NPAD, EMB), jnp.float32),
                   jax.ShapeDtypeStruct((8, EMB), jnp.float32),
                   jax.ShapeDtypeStruct((8, EMB), jnp.float32)),
    )(sp, sp, g, dinvb)


def _tc_mid(agg, ssum, ssq, gam, bet, w, bias, dinvb):
    def body(agg_ref, sum_ref, ssq_ref, gam_ref, bet_ref, w_ref, b_ref,
             dv_ref, o_ref):
        inv_n = 1.0 / N_NODES
        mean = sum_ref[0:1, :] * inv_n
        var = ssq_ref[0:1, :] * inv_n - mean * mean
        a = gam_ref[...] * lax.rsqrt(var + BN_EPS)
        csh = bet_ref[...] - mean * a
        u = jnp.maximum(agg_ref[...] * a + csh, 0.0)
        h = jnp.dot(u, w_ref[...], preferred_element_type=jnp.float32)
        o_ref[...] = dv_ref[...] * (h + b_ref[...])

    return pl.pallas_call(
        body,
        grid=(NBLK,),
        in_specs=[pl.BlockSpec((128, EMB), lambda i: (i, 0)),
                  pl.BlockSpec((8, EMB), lambda i: (0, 0)),
                  pl.BlockSpec((8, EMB), lambda i: (0, 0)),
                  pl.BlockSpec((1, EMB), lambda i: (0, 0)),
                  pl.BlockSpec((1, EMB), lambda i: (0, 0)),
                  pl.BlockSpec((EMB, EMB), lambda i: (0, 0)),
                  pl.BlockSpec((1, EMB), lambda i: (0, 0)),
                  pl.BlockSpec((128, EMB), lambda i: (i, 0))],
        out_specs=pl.BlockSpec((128, EMB), lambda i: (i, 0)),
        out_shape=jax.ShapeDtypeStruct((NPAD, EMB), jnp.float32),
    )(agg, ssum, ssq, gam, bet, w, bias, dinvb)


def _tc_last(agg, ssum, ssq, gam, bet):
    def body(agg_ref, sum_ref, ssq_ref, gam_ref, bet_ref, o_ref):
        inv_n = 1.0 / N_NODES
        mean = sum_ref[0:1, :] * inv_n
        var = ssq_ref[0:1, :] * inv_n - mean * mean
        a = gam_ref[...] * lax.rsqrt(var + BN_EPS)
        csh = bet_ref[...] - mean * a
        o_ref[...] = agg_ref[...] * a + csh

    return pl.pallas_call(
        body,
        grid=(NBLK,),
        in_specs=[pl.BlockSpec((128, EMB), lambda i: (i, 0)),
                  pl.BlockSpec((8, EMB), lambda i: (0, 0)),
                  pl.BlockSpec((8, EMB), lambda i: (0, 0)),
                  pl.BlockSpec((1, EMB), lambda i: (0, 0)),
                  pl.BlockSpec((1, EMB), lambda i: (0, 0))],
        out_specs=pl.BlockSpec((128, EMB), lambda i: (i, 0)),
        out_shape=jax.ShapeDtypeStruct((NPAD, EMB), jnp.float32),
    )(agg, ssum, ssq, gam, bet)


def _tc_pool_div(psum, cnt):
    def body(p0_ref, p1_ref, c0_ref, c1_ref, o_ref):
        cc = c0_ref[0, :, 0:1] + c1_ref[0, :, 0:1]
        o_ref[...] = (p0_ref[0] + p1_ref[0]) / jnp.maximum(cc, 1.0)

    return pl.pallas_call(
        body,
        grid=(1,),
        in_specs=[pl.BlockSpec((1, NUM_GRAPHS, EMB), lambda i: (0, 0, 0)),
                  pl.BlockSpec((1, NUM_GRAPHS, EMB), lambda i: (1, 0, 0)),
                  pl.BlockSpec((1, NUM_GRAPHS, 16), lambda i: (0, 0, 0)),
                  pl.BlockSpec((1, NUM_GRAPHS, 16), lambda i: (1, 0, 0))],
        out_specs=pl.BlockSpec((NUM_GRAPHS, EMB), lambda i: (0, 0)),
        out_shape=jax.ShapeDtypeStruct((NUM_GRAPHS, EMB), jnp.float32),
    )(psum, psum, cnt, cnt)


def kernel(x, edge_index, edge_attr, batch, W, b, gamma, beta):
    del edge_attr
    f32 = jnp.float32

    src = edge_index[0].astype(jnp.int32)
    dst = edge_index[1].astype(jnp.int32)
    packed = src + dst * 16384
    sentinel = jnp.int32(PAD_ROW + PAD_ROW * 16384)
    fast = packed[:E_FAST].reshape(NS, NCHF, CK)
    slow = jnp.concatenate(
        [packed[E_FAST:],
         jnp.full((NS * NCHS * CK - E_SLOW,), sentinel, jnp.int32)]
    ).reshape(NS, NCHS, CK)
    slow = jnp.concatenate(
        [slow, jnp.full((NS, NCHM - NCHS, CK), sentinel, jnp.int32)],
        axis=1)
    cores = [None, None]
    cores[FAST_C] = fast
    cores[1 - FAST_C] = slow
    epack = jnp.stack(cores, axis=1).reshape(NW, NCHM, CK)
    batch3 = jnp.concatenate(
        [batch.astype(jnp.int32),
         jnp.full((NPAD - N_NODES,), NUM_GRAPHS, dtype=jnp.int32)]
    ).reshape(NW, NBCH, BK)
    x_pad = jnp.concatenate(
        [x.astype(f32), jnp.zeros((NPAD - N_NODES, EMB), f32)], axis=0)

    degp = _sc_degree(epack)
    dinvb = _tc_dinv(degp)
    g = _tc_entry(x_pad, W[0], b[0].reshape(1, EMB), dinvb)

    h_final = None
    for l in range(NUM_LAYER):
        sp = _sc_scatter(g, epack)
        agg, ssum, ssq = _tc_agg_stats(sp, g, dinvb)
        gam = gamma[l].reshape(1, EMB)
        bet = beta[l].reshape(1, EMB)
        if l < NUM_LAYER - 1:
            g = _tc_mid(agg, ssum, ssq, gam, bet,
                        W[l + 1], b[l + 1].reshape(1, EMB), dinvb)
        else:
            h_final = _tc_last(agg, ssum, ssq, gam, bet)

    psum, cnt = _sc_pool(h_final, batch3)
    return _tc_pool_div(psum, cnt)


# speedup vs baseline: 2.6174x; 1.2629x over previous
import jax
import jax.numpy as jnp
from jax import lax
from jax.experimental import pallas as pl
from jax.experimental.pallas import tpu as pltpu
from jax.experimental.pallas import tpu_sc as plsc

N_NODES = 10000
EMB = 128
NUM_LAYER = 5
NUM_GRAPHS = 512
BN_EPS = 1e-5

NC = 2
NS = 16
NW = NC * NS

NPAD = 10240
PAD_ROW = N_NODES
NBLK = NPAD // 128

E_TOTAL = 320000
CK = 112
FAST_C = 1
NCHF = 138
NCHS = 42
NCHM = NCHF
E_FAST = NS * NCHF * CK
E_SLOW = E_TOTAL - E_FAST
ROWS_PER_TILE = NPAD // NS
ACC_R = 10112
ACC_PT = ACC_R // NS

PG = 640
PBR = PG // NS
BK = 64
BPT = NPAD // NW
NBCH = BPT // BK


def _sc_mesh():
    return plsc.VectorSubcoreMesh(
        core_axis_name="c", subcore_axis_name="s",
        num_cores=NC, num_subcores=NS)


def _fill_2d(ref, rows, cols, value):
    v = jnp.full((16,), value, jnp.float32)
    cpr = cols // 16

    def st(t, carry):
        ref[t // cpr, pl.ds((t % cpr) * 16, 16)] = v
        return carry

    lax.fori_loop(0, rows * cpr, st, 0)


def _unpack_chunk(packed, sidx_c, didx_c, r, k):
    def st(t, carry):
        v = packed[k, pl.ds(t * 16, 16)]
        sidx_c[r, pl.ds(t * 16, 16)] = lax.bitwise_and(v, 16383)
        didx_c[r, pl.ds(t * 16, 16)] = lax.shift_right_logical(v, 14)
        return carry

    lax.fori_loop(0, CK // 16, st, 0)


def _sc_degree(epack):
    def body(e_hbm, out_hbm, idx_v, ones_v, zero_v, accum):
        c = lax.axis_index("c")
        s = lax.axis_index("s")
        wid = s * NC + c
        _fill_2d(ones_v, CK, 16, 1.0)
        _fill_2d(zero_v, CK, 16, 0.0)
        pltpu.sync_copy(e_hbm.at[wid], idx_v)
        cpr = CK // 16

        def st(t, carry):
            r = t // cpr
            o = (t % cpr) * 16
            idx_v[r, pl.ds(o, 16)] = lax.shift_right_logical(
                idx_v[r, pl.ds(o, 16)], 14)
            return carry

        lax.fori_loop(0, NCHM * cpr, st, 0)
        base = s * ROWS_PER_TILE
        for r in range(ROWS_PER_TILE // CK):
            pltpu.sync_copy(zero_v, accum.at[pl.ds(base + r * CK, CK)])
        zrem = ROWS_PER_TILE % CK
        if zrem:
            pltpu.sync_copy(
                zero_v.at[pl.ds(0, zrem)],
                accum.at[pl.ds(base + ROWS_PER_TILE - zrem, zrem)])
        plsc.subcore_barrier()

        def chunk(j, carry):
            pltpu.sync_copy(ones_v, accum.at[idx_v.at[j]], add=True)
            return carry

        lax.fori_loop(0, NCHM, chunk, 0)
        plsc.subcore_barrier()
        pltpu.sync_copy(accum.at[pl.ds(base, ROWS_PER_TILE)],
                        out_hbm.at[c, pl.ds(base, ROWS_PER_TILE)])

    return pl.kernel(
        body,
        out_type=jax.ShapeDtypeStruct((NC, NPAD, 16), jnp.float32),
        mesh=_sc_mesh(),
        scratch_types=[
            pltpu.VMEM((NCHM, CK), jnp.int32),
            pltpu.VMEM((CK, 16), jnp.float32),
            pltpu.VMEM((CK, 16), jnp.float32),
            pltpu.VMEM_SHARED((NPAD, 16), jnp.float32),
        ],
    )(epack)


def _sc_scatter(g, epack):
    def body(g_hbm, e_hbm, out_hbm, packed, sidx_c, didx_c, bufs, accum,
             sem):
        c = lax.axis_index("c")
        s = lax.axis_index("s")
        wid = s * NC + c
        pltpu.sync_copy(e_hbm.at[wid], packed)

        def zr(t, carry):
            bufs[0, t // 8, pl.ds((t % 8) * 16, 16)] = jnp.zeros(
                (16,), jnp.float32)
            return carry

        lax.fori_loop(0, CK * 8, zr, 0)
        base = s * ACC_PT
        for r in range(ACC_PT // CK):
            pltpu.sync_copy(bufs.at[0], accum.at[pl.ds(base + r * CK, CK)])
        rem = ACC_PT % CK
        pltpu.sync_copy(bufs.at[0, pl.ds(0, rem)],
                        accum.at[pl.ds(base + ACC_PT - rem, rem)])
        plsc.subcore_barrier()

        def issue_gather(r):
            pltpu.async_copy(g_hbm.at[sidx_c.at[r]], bufs.at[r], sem)

        def wait_gather(r):
            pltpu.make_async_copy(
                g_hbm.at[sidx_c.at[r]], bufs.at[r], sem).wait()

        _unpack_chunk(packed, sidx_c, didx_c, 0, 0)
        issue_gather(0)

        def make_pair(n_half):
            def pair(j, carry):
                e = 2 * j
                _unpack_chunk(packed, sidx_c, didx_c, 1, e + 1)
                wait_gather(0)
                issue_gather(1)
                pltpu.sync_copy(bufs.at[0], accum.at[didx_c.at[0]],
                                add=True)
                _unpack_chunk(packed, sidx_c, didx_c, 0,
                              jnp.minimum(e + 2, NCHM - 1))
                wait_gather(1)

                @pl.when(j < n_half - 1)
                def _():
                    issue_gather(0)

                pltpu.sync_copy(bufs.at[1], accum.at[didx_c.at[1]],
                                add=True)
                return carry

            return pair

        @pl.when(c == FAST_C)
        def _():
            lax.fori_loop(0, NCHF // 2, make_pair(NCHF // 2), 0)

        @pl.when(c != FAST_C)
        def _():
            lax.fori_loop(0, NCHS // 2, make_pair(NCHS // 2), 0)

        plsc.subcore_barrier()
        pltpu.sync_copy(accum.at[pl.ds(base, ACC_PT)],
                        out_hbm.at[c, pl.ds(base, ACC_PT)])

    return pl.kernel(
        body,
        out_type=jax.ShapeDtypeStruct((NC, NPAD, EMB), jnp.float32),
        mesh=_sc_mesh(),
        scratch_types=[
            pltpu.VMEM((NCHM, CK), jnp.int32),
            pltpu.VMEM((8, CK), jnp.int32),
            pltpu.VMEM((8, CK), jnp.int32),
            pltpu.VMEM((2, CK, EMB), jnp.float32),
            pltpu.VMEM_SHARED((ACC_R, EMB), jnp.float32),
            pltpu.SemaphoreType.DMA,
        ],
    )(g, epack)


def _sc_pool(h, batch3):
    def body(h_hbm, b_hbm, psum_hbm, cnt_hbm,
             bidx, rows_v, ones_v, zp_v, zc_v, paccum, caccum):
        c = lax.axis_index("c")
        s = lax.axis_index("s")
        wid = s * NC + c
        _fill_2d(ones_v, BK, 16, 1.0)
        _fill_2d(zp_v, PBR, EMB, 0.0)
        _fill_2d(zc_v, PBR, 16, 0.0)
        pltpu.sync_copy(b_hbm.at[wid], bidx)
        pltpu.sync_copy(zp_v, paccum.at[pl.ds(s * PBR, PBR)])
        pltpu.sync_copy(zc_v, caccum.at[pl.ds(s * PBR, PBR)])
        plsc.subcore_barrier()

        def chunk(t, carry):
            pltpu.sync_copy(h_hbm.at[pl.ds(wid * BPT + t * BK, BK)], rows_v)
            pltpu.sync_copy(rows_v, paccum.at[bidx.at[t]], add=True)
            pltpu.sync_copy(ones_v, caccum.at[bidx.at[t]], add=True)
            return carry

        lax.fori_loop(0, NBCH, chunk, 0)
        plsc.subcore_barrier()
        pltpu.sync_copy(paccum.at[pl.ds(s * PBR, PBR)],
                        psum_hbm.at[c, pl.ds(s * PBR, PBR)])
        pltpu.sync_copy(caccum.at[pl.ds(s * PBR, PBR)],
                        cnt_hbm.at[c, pl.ds(s * PBR, PBR)])

    return pl.kernel(
        body,
        out_type=(jax.ShapeDtypeStruct((NC, PG, EMB), jnp.float32),
                  jax.ShapeDtypeStruct((NC, PG, 16), jnp.float32)),
        mesh=_sc_mesh(),
        scratch_types=[
            pltpu.VMEM((NBCH, BK), jnp.int32),
            pltpu.VMEM((BK, EMB), jnp.float32),
            pltpu.VMEM((BK, 16), jnp.float32),
            pltpu.VMEM((PBR, EMB), jnp.float32),
            pltpu.VMEM((PBR, 16), jnp.float32),
            pltpu.VMEM_SHARED((PG, EMB), jnp.float32),
            pltpu.VMEM_SHARED((PG, 16), jnp.float32),
        ],
    )(h, batch3)


def _tc_dinv(degp):
    def body(d0_ref, d1_ref, o_ref):
        i = pl.program_id(0)
        deg = 1.0 + d0_ref[0, :, 0:1] + d1_ref[0, :, 0:1]
        dinv = lax.rsqrt(deg)
        rid = i * TCB + lax.broadcasted_iota(jnp.int32, (TCB, 1), 0)
        dinv = jnp.where(rid < N_NODES, dinv, 0.0)
        o_ref[...] = jnp.broadcast_to(dinv, (TCB, EMB))

    return pl.pallas_call(
        body,
        grid=(TNB,),
        in_specs=[pl.BlockSpec((1, TCB, 16), lambda i: (0, i, 0)),
                  pl.BlockSpec((1, TCB, 16), lambda i: (1, i, 0))],
        out_specs=pl.BlockSpec((TCB, EMB), lambda i: (i, 0)),
        out_shape=jax.ShapeDtypeStruct((NPAD, EMB), jnp.float32),
    )(degp, degp)


def _tc_entry(x, w, bias, dinvb):
    def body(x_ref, w_ref, b_ref, dv_ref, o_ref):
        h = jnp.dot(x_ref[...], w_ref[...], preferred_element_type=jnp.float32)
        o_ref[...] = dv_ref[...] * (h + b_ref[...])

    return pl.pallas_call(
        body,
        grid=(TNB,),
        in_specs=[pl.BlockSpec((TCB, EMB), lambda i: (i, 0)),
                  pl.BlockSpec((EMB, EMB), lambda i: (0, 0)),
                  pl.BlockSpec((1, EMB), lambda i: (0, 0)),
                  pl.BlockSpec((TCB, EMB), lambda i: (i, 0))],
        out_specs=pl.BlockSpec((TCB, EMB), lambda i: (i, 0)),
        out_shape=jax.ShapeDtypeStruct((NPAD, EMB), jnp.float32),
    )(x, w, bias, dinvb)


TCB = 512
TNB = NPAD // TCB


def _tc_layer(sp, g, dinvb, gam, bet, w, bias, last):
    def body(s0_ref, s1_ref, g_ref, dv_ref, gam_ref, bet_ref, w_ref,
             b_ref, o_ref, agg_s, sum_s, ssq_s):
        p = pl.program_id(0)
        i = pl.program_id(1)

        @pl.when(p == 0)
        def _():
            agg = dv_ref[...] * (s0_ref[0] + s1_ref[0] + g_ref[...])
            rid = i * TCB + lax.broadcasted_iota(jnp.int32, (TCB, 1), 0)
            agg = jnp.where(rid < N_NODES, agg, 0.0)
            agg_s[pl.ds(i * TCB, TCB), :] = agg

            @pl.when(i == 0)
            def _():
                sum_s[...] = jnp.zeros_like(sum_s)
                ssq_s[...] = jnp.zeros_like(ssq_s)

            sum_s[...] += jnp.broadcast_to(
                jnp.sum(agg, axis=0, keepdims=True), (8, EMB))
            ssq_s[...] += jnp.broadcast_to(
                jnp.sum(agg * agg, axis=0, keepdims=True), (8, EMB))

        @pl.when(p == 1)
        def _():
            inv_n = 1.0 / N_NODES
            mean = sum_s[0:1, :] * inv_n
            var = ssq_s[0:1, :] * inv_n - mean * mean
            a = gam_ref[...] * lax.rsqrt(var + BN_EPS)
            csh = bet_ref[...] - mean * a
            agg = agg_s[pl.ds(i * TCB, TCB), :]
            if last:
                o_ref[...] = agg * a + csh
            else:
                u = jnp.maximum(agg * a + csh, 0.0)
                h = jnp.dot(u, w_ref[...],
                            preferred_element_type=jnp.float32)
                o_ref[...] = dv_ref[...] * (h + b_ref[...])

    return pl.pallas_call(
        body,
        grid=(2, TNB),
        in_specs=[
            pl.BlockSpec((1, TCB, EMB), lambda p, i: (0, i * (1 - p), 0)),
            pl.BlockSpec((1, TCB, EMB), lambda p, i: (1, i * (1 - p), 0)),
            pl.BlockSpec((TCB, EMB), lambda p, i: (i * (1 - p), 0)),
            pl.BlockSpec((TCB, EMB), lambda p, i: (i, 0)),
            pl.BlockSpec((1, EMB), lambda p, i: (0, 0)),
            pl.BlockSpec((1, EMB), lambda p, i: (0, 0)),
            pl.BlockSpec((EMB, EMB), lambda p, i: (0, 0)),
            pl.BlockSpec((1, EMB), lambda p, i: (0, 0)),
        ],
        out_specs=pl.BlockSpec((TCB, EMB), lambda p, i: (i, 0)),
        out_shape=jax.ShapeDtypeStruct((NPAD, EMB), jnp.float32),
        scratch_shapes=[pltpu.VMEM((NPAD, EMB), jnp.float32),
                        pltpu.VMEM((8, EMB), jnp.float32),
                        pltpu.VMEM((8, EMB), jnp.float32)],
    )(sp, sp, g, dinvb, gam, bet, w, bias)


def _tc_pool_div(psum, cnt):
    def body(p0_ref, p1_ref, c0_ref, c1_ref, o_ref):
        cc = c0_ref[0, :, 0:1] + c1_ref[0, :, 0:1]
        o_ref[...] = (p0_ref[0] + p1_ref[0]) / jnp.maximum(cc, 1.0)

    return pl.pallas_call(
        body,
        grid=(1,),
        in_specs=[pl.BlockSpec((1, NUM_GRAPHS, EMB), lambda i: (0, 0, 0)),
                  pl.BlockSpec((1, NUM_GRAPHS, EMB), lambda i: (1, 0, 0)),
                  pl.BlockSpec((1, NUM_GRAPHS, 16), lambda i: (0, 0, 0)),
                  pl.BlockSpec((1, NUM_GRAPHS, 16), lambda i: (1, 0, 0))],
        out_specs=pl.BlockSpec((NUM_GRAPHS, EMB), lambda i: (0, 0)),
        out_shape=jax.ShapeDtypeStruct((NUM_GRAPHS, EMB), jnp.float32),
    )(psum, psum, cnt, cnt)


def kernel(x, edge_index, edge_attr, batch, W, b, gamma, beta):
    del edge_attr
    f32 = jnp.float32

    src = edge_index[0].astype(jnp.int32)
    dst = edge_index[1].astype(jnp.int32)
    packed = src + dst * 16384
    sentinel = jnp.int32(PAD_ROW + PAD_ROW * 16384)
    fast = packed[:E_FAST].reshape(NS, NCHF, CK)
    slow = jnp.concatenate(
        [packed[E_FAST:],
         jnp.full((NS * NCHS * CK - E_SLOW,), sentinel, jnp.int32)]
    ).reshape(NS, NCHS, CK)
    slow = jnp.concatenate(
        [slow, jnp.full((NS, NCHM - NCHS, CK), sentinel, jnp.int32)],
        axis=1)
    cores = [None, None]
    cores[FAST_C] = fast
    cores[1 - FAST_C] = slow
    epack = jnp.stack(cores, axis=1).reshape(NW, NCHM, CK)
    batch3 = jnp.concatenate(
        [batch.astype(jnp.int32),
         jnp.full((NPAD - N_NODES,), NUM_GRAPHS, dtype=jnp.int32)]
    ).reshape(NW, NBCH, BK)
    x_pad = jnp.concatenate(
        [x.astype(f32), jnp.zeros((NPAD - N_NODES, EMB), f32)], axis=0)

    degp = _sc_degree(epack)
    dinvb = _tc_dinv(degp)
    g = _tc_entry(x_pad, W[0], b[0].reshape(1, EMB), dinvb)

    h_final = None
    for l in range(NUM_LAYER):
        sp = _sc_scatter(g, epack)
        gam = gamma[l].reshape(1, EMB)
        bet = beta[l].reshape(1, EMB)
        last = l == NUM_LAYER - 1
        wn = W[0] if last else W[l + 1]
        bn = (b[0] if last else b[l + 1]).reshape(1, EMB)
        out = _tc_layer(sp, g, dinvb, gam, bet, wn, bn, last)
        if last:
            h_final = out
        else:
            g = out

    psum, cnt = _sc_pool(h_final, batch3)
    return _tc_pool_div(psum, cnt)


# speedup vs baseline: 2.9805x; 1.1387x over previous
import jax
import jax.numpy as jnp
from jax import lax
from jax.experimental import pallas as pl
from jax.experimental.pallas import tpu as pltpu
from jax.experimental.pallas import tpu_sc as plsc

N_NODES = 10000
EMB = 128
NUM_LAYER = 5
NUM_GRAPHS = 512
BN_EPS = 1e-5

NC = 2
NS = 16
NW = NC * NS

NPAD = 10240
PAD_ROW = N_NODES
NBLK = NPAD // 128

E_TOTAL = 320000
CK = 112
FAST_C = 1
NCHF = 138
NCHS = 42
NCHM = NCHF
E_FAST = NS * NCHF * CK
E_SLOW = E_TOTAL - E_FAST
DNCH = 90
ROWS_PER_TILE = NPAD // NS
ACC_R = 10112
ACC_PT = ACC_R // NS

PG = 640
PBR = PG // NS
BK = 64
BPT = NPAD // NW
NBCH = BPT // BK


def _sc_mesh():
    return plsc.VectorSubcoreMesh(
        core_axis_name="c", subcore_axis_name="s",
        num_cores=NC, num_subcores=NS)


def _fill_2d(ref, rows, cols, value):
    v = jnp.full((16,), value, jnp.float32)
    cpr = cols // 16

    def st(t, carry):
        ref[t // cpr, pl.ds((t % cpr) * 16, 16)] = v
        return carry

    lax.fori_loop(0, rows * cpr, st, 0)


def _unpack_chunk(packed, sidx_c, didx_c, r, k):
    def st(t, carry):
        v = packed[k, pl.ds(t * 16, 16)]
        sidx_c[r, pl.ds(t * 16, 16)] = lax.bitwise_and(v, 16383)
        didx_c[r, pl.ds(t * 16, 16)] = lax.shift_right_logical(v, 14)
        return carry

    lax.fori_loop(0, CK // 16, st, 0)


def _sc_degree(epack):
    def body(e_hbm, out_hbm, idx_v, ones_v, zero_v, accum):
        c = lax.axis_index("c")
        s = lax.axis_index("s")
        wid = s * NC + c
        _fill_2d(ones_v, CK, 16, 1.0)
        _fill_2d(zero_v, CK, 16, 0.0)
        pltpu.sync_copy(e_hbm.at[wid], idx_v)
        cpr = CK // 16

        def st(t, carry):
            r = t // cpr
            o = (t % cpr) * 16
            idx_v[r, pl.ds(o, 16)] = lax.shift_right_logical(
                idx_v[r, pl.ds(o, 16)], 14)
            return carry

        lax.fori_loop(0, DNCH * cpr, st, 0)
        base = s * ROWS_PER_TILE
        for r in range(ROWS_PER_TILE // CK):
            pltpu.sync_copy(zero_v, accum.at[pl.ds(base + r * CK, CK)])
        zrem = ROWS_PER_TILE % CK
        if zrem:
            pltpu.sync_copy(
                zero_v.at[pl.ds(0, zrem)],
                accum.at[pl.ds(base + ROWS_PER_TILE - zrem, zrem)])
        plsc.subcore_barrier()

        def chunk(j, carry):
            pltpu.sync_copy(ones_v, accum.at[idx_v.at[j]], add=True)
            return carry

        lax.fori_loop(0, DNCH, chunk, 0)
        plsc.subcore_barrier()
        pltpu.sync_copy(accum.at[pl.ds(base, ROWS_PER_TILE)],
                        out_hbm.at[c, pl.ds(base, ROWS_PER_TILE)])

    return pl.kernel(
        body,
        out_type=jax.ShapeDtypeStruct((NC, NPAD, 16), jnp.float32),
        mesh=_sc_mesh(),
        scratch_types=[
            pltpu.VMEM((DNCH, CK), jnp.int32),
            pltpu.VMEM((CK, 16), jnp.float32),
            pltpu.VMEM((CK, 16), jnp.float32),
            pltpu.VMEM_SHARED((NPAD, 16), jnp.float32),
        ],
    )(epack)


def _sc_scatter(g, epack):
    def body(g_hbm, e_hbm, out_hbm, packed, sidx_c, didx_c, bufs, accum,
             sem):
        c = lax.axis_index("c")
        s = lax.axis_index("s")
        wid = s * NC + c
        pltpu.sync_copy(e_hbm.at[wid], packed)

        def zr(t, carry):
            bufs[0, t // 8, pl.ds((t % 8) * 16, 16)] = jnp.zeros(
                (16,), jnp.float32)
            return carry

        lax.fori_loop(0, CK * 8, zr, 0)
        base = s * ACC_PT
        for r in range(ACC_PT // CK):
            pltpu.sync_copy(bufs.at[0], accum.at[pl.ds(base + r * CK, CK)])
        rem = ACC_PT % CK
        pltpu.sync_copy(bufs.at[0, pl.ds(0, rem)],
                        accum.at[pl.ds(base + ACC_PT - rem, rem)])
        plsc.subcore_barrier()

        def issue_gather(r):
            pltpu.async_copy(g_hbm.at[sidx_c.at[r]], bufs.at[r], sem)

        def wait_gather(r):
            pltpu.make_async_copy(
                g_hbm.at[sidx_c.at[r]], bufs.at[r], sem).wait()

        _unpack_chunk(packed, sidx_c, didx_c, 0, 0)
        issue_gather(0)

        def make_pair(n_half):
            def pair(j, carry):
                e = 2 * j
                _unpack_chunk(packed, sidx_c, didx_c, 1, e + 1)
                wait_gather(0)
                issue_gather(1)
                pltpu.sync_copy(bufs.at[0], accum.at[didx_c.at[0]],
                                add=True)
                _unpack_chunk(packed, sidx_c, didx_c, 0,
                              jnp.minimum(e + 2, NCHM - 1))
                wait_gather(1)

                @pl.when(j < n_half - 1)
                def _():
                    issue_gather(0)

                pltpu.sync_copy(bufs.at[1], accum.at[didx_c.at[1]],
                                add=True)
                return carry

            return pair

        @pl.when(c == FAST_C)
        def _():
            lax.fori_loop(0, NCHF // 2, make_pair(NCHF // 2), 0)

        @pl.when(c != FAST_C)
        def _():
            lax.fori_loop(0, NCHS // 2, make_pair(NCHS // 2), 0)

        plsc.subcore_barrier()
        pltpu.sync_copy(accum.at[pl.ds(base, ACC_PT)],
                        out_hbm.at[c, pl.ds(base, ACC_PT)])

    return pl.kernel(
        body,
        out_type=jax.ShapeDtypeStruct((NC, NPAD, EMB), jnp.float32),
        mesh=_sc_mesh(),
        scratch_types=[
            pltpu.VMEM((NCHM, CK), jnp.int32),
            pltpu.VMEM((8, CK), jnp.int32),
            pltpu.VMEM((8, CK), jnp.int32),
            pltpu.VMEM((2, CK, EMB), jnp.float32),
            pltpu.VMEM_SHARED((ACC_R, EMB), jnp.float32),
            pltpu.SemaphoreType.DMA,
        ],
    )(g, epack)


def _sc_pool(h, batch3):
    def body(h_hbm, b_hbm, psum_hbm, cnt_hbm,
             bidx, rows_v, ones_v, zp_v, zc_v, paccum, caccum):
        c = lax.axis_index("c")
        s = lax.axis_index("s")
        wid = s * NC + c
        _fill_2d(ones_v, BK, 16, 1.0)
        _fill_2d(zp_v, PBR, EMB, 0.0)
        _fill_2d(zc_v, PBR, 16, 0.0)
        pltpu.sync_copy(b_hbm.at[wid], bidx)
        pltpu.sync_copy(zp_v, paccum.at[pl.ds(s * PBR, PBR)])
        pltpu.sync_copy(zc_v, caccum.at[pl.ds(s * PBR, PBR)])
        plsc.subcore_barrier()

        def chunk(t, carry):
            pltpu.sync_copy(h_hbm.at[pl.ds(wid * BPT + t * BK, BK)], rows_v)
            pltpu.sync_copy(rows_v, paccum.at[bidx.at[t]], add=True)
            pltpu.sync_copy(ones_v, caccum.at[bidx.at[t]], add=True)
            return carry

        lax.fori_loop(0, NBCH, chunk, 0)
        plsc.subcore_barrier()
        pltpu.sync_copy(paccum.at[pl.ds(s * PBR, PBR)],
                        psum_hbm.at[c, pl.ds(s * PBR, PBR)])
        pltpu.sync_copy(caccum.at[pl.ds(s * PBR, PBR)],
                        cnt_hbm.at[c, pl.ds(s * PBR, PBR)])

    return pl.kernel(
        body,
        out_type=(jax.ShapeDtypeStruct((NC, PG, EMB), jnp.float32),
                  jax.ShapeDtypeStruct((NC, PG, 16), jnp.float32)),
        mesh=_sc_mesh(),
        scratch_types=[
            pltpu.VMEM((NBCH, BK), jnp.int32),
            pltpu.VMEM((BK, EMB), jnp.float32),
            pltpu.VMEM((BK, 16), jnp.float32),
            pltpu.VMEM((PBR, EMB), jnp.float32),
            pltpu.VMEM((PBR, 16), jnp.float32),
            pltpu.VMEM_SHARED((PG, EMB), jnp.float32),
            pltpu.VMEM_SHARED((PG, 16), jnp.float32),
        ],
    )(h, batch3)


def _tc_dinv(degp):
    def body(d0_ref, d1_ref, o_ref):
        i = pl.program_id(0)
        deg = 1.0 + d0_ref[0, :, 0:1] + d1_ref[0, :, 0:1]
        dinv = lax.rsqrt(deg)
        rid = i * TCB + lax.broadcasted_iota(jnp.int32, (TCB, 1), 0)
        dinv = jnp.where(rid < N_NODES, dinv, 0.0)
        o_ref[...] = jnp.broadcast_to(dinv, (TCB, EMB))

    return pl.pallas_call(
        body,
        grid=(TNB,),
        in_specs=[pl.BlockSpec((1, TCB, 16), lambda i: (0, i, 0)),
                  pl.BlockSpec((1, TCB, 16), lambda i: (1, i, 0))],
        out_specs=pl.BlockSpec((TCB, EMB), lambda i: (i, 0)),
        out_shape=jax.ShapeDtypeStruct((NPAD, EMB), jnp.float32),
    )(degp, degp)


def _tc_entry(x, w, bias, dinvb):
    def body(x_ref, w_ref, b_ref, dv_ref, o_ref):
        h = jnp.dot(x_ref[...], w_ref[...], preferred_element_type=jnp.float32)
        o_ref[...] = dv_ref[...] * (h + b_ref[...])

    return pl.pallas_call(
        body,
        grid=(TNB,),
        in_specs=[pl.BlockSpec((TCB, EMB), lambda i: (i, 0)),
                  pl.BlockSpec((EMB, EMB), lambda i: (0, 0)),
                  pl.BlockSpec((1, EMB), lambda i: (0, 0)),
                  pl.BlockSpec((TCB, EMB), lambda i: (i, 0))],
        out_specs=pl.BlockSpec((TCB, EMB), lambda i: (i, 0)),
        out_shape=jax.ShapeDtypeStruct((NPAD, EMB), jnp.float32),
    )(x, w, bias, dinvb)


TCB = 512
TNB = NPAD // TCB


def _tc_layer(sp, g, dinvb, gam, bet, w, bias, last):
    def body(s0_ref, s1_ref, g_ref, dv_ref, gam_ref, bet_ref, w_ref,
             b_ref, o_ref, agg_s, sum_s, ssq_s):
        p = pl.program_id(0)
        i = pl.program_id(1)

        @pl.when(p == 0)
        def _():
            agg = dv_ref[...] * (s0_ref[0] + s1_ref[0] + g_ref[...])
            rid = i * TCB + lax.broadcasted_iota(jnp.int32, (TCB, 1), 0)
            agg = jnp.where(rid < N_NODES, agg, 0.0)
            agg_s[pl.ds(i * TCB, TCB), :] = agg

            @pl.when(i == 0)
            def _():
                sum_s[...] = jnp.zeros_like(sum_s)
                ssq_s[...] = jnp.zeros_like(ssq_s)

            sum_s[...] += jnp.broadcast_to(
                jnp.sum(agg, axis=0, keepdims=True), (8, EMB))
            ssq_s[...] += jnp.broadcast_to(
                jnp.sum(agg * agg, axis=0, keepdims=True), (8, EMB))

        @pl.when(p == 1)
        def _():
            inv_n = 1.0 / N_NODES
            mean = sum_s[0:1, :] * inv_n
            var = ssq_s[0:1, :] * inv_n - mean * mean
            a = gam_ref[...] * lax.rsqrt(var + BN_EPS)
            csh = bet_ref[...] - mean * a
            agg = agg_s[pl.ds(i * TCB, TCB), :]
            if last:
                o_ref[...] = agg * a + csh
            else:
                u = jnp.maximum(agg * a + csh, 0.0)
                h = jnp.dot(u, w_ref[...],
                            preferred_element_type=jnp.float32)
                o_ref[...] = dv_ref[...] * (h + b_ref[...])

    return pl.pallas_call(
        body,
        grid=(2, TNB),
        in_specs=[
            pl.BlockSpec((1, TCB, EMB), lambda p, i: (0, i * (1 - p), 0)),
            pl.BlockSpec((1, TCB, EMB), lambda p, i: (1, i * (1 - p), 0)),
            pl.BlockSpec((TCB, EMB), lambda p, i: (i * (1 - p), 0)),
            pl.BlockSpec((TCB, EMB), lambda p, i: (i, 0)),
            pl.BlockSpec((1, EMB), lambda p, i: (0, 0)),
            pl.BlockSpec((1, EMB), lambda p, i: (0, 0)),
            pl.BlockSpec((EMB, EMB), lambda p, i: (0, 0)),
            pl.BlockSpec((1, EMB), lambda p, i: (0, 0)),
        ],
        out_specs=pl.BlockSpec((TCB, EMB), lambda p, i: (i, 0)),
        out_shape=jax.ShapeDtypeStruct((NPAD, EMB), jnp.float32),
        scratch_shapes=[pltpu.VMEM((NPAD, EMB), jnp.float32),
                        pltpu.VMEM((8, EMB), jnp.float32),
                        pltpu.VMEM((8, EMB), jnp.float32)],
    )(sp, sp, g, dinvb, gam, bet, w, bias)


def _tc_pool_div(psum, cnt):
    def body(p0_ref, p1_ref, c0_ref, c1_ref, o_ref):
        cc = c0_ref[0, :, 0:1] + c1_ref[0, :, 0:1]
        o_ref[...] = (p0_ref[0] + p1_ref[0]) / jnp.maximum(cc, 1.0)

    return pl.pallas_call(
        body,
        grid=(1,),
        in_specs=[pl.BlockSpec((1, NUM_GRAPHS, EMB), lambda i: (0, 0, 0)),
                  pl.BlockSpec((1, NUM_GRAPHS, EMB), lambda i: (1, 0, 0)),
                  pl.BlockSpec((1, NUM_GRAPHS, 16), lambda i: (0, 0, 0)),
                  pl.BlockSpec((1, NUM_GRAPHS, 16), lambda i: (1, 0, 0))],
        out_specs=pl.BlockSpec((NUM_GRAPHS, EMB), lambda i: (0, 0)),
        out_shape=jax.ShapeDtypeStruct((NUM_GRAPHS, EMB), jnp.float32),
    )(psum, psum, cnt, cnt)


def kernel(x, edge_index, edge_attr, batch, W, b, gamma, beta):
    del edge_attr
    f32 = jnp.float32

    src = edge_index[0].astype(jnp.int32)
    dst = edge_index[1].astype(jnp.int32)
    packed = src + dst * 16384
    sentinel = jnp.int32(PAD_ROW + PAD_ROW * 16384)
    fast = packed[:E_FAST].reshape(NS, NCHF, CK)
    slow = jnp.concatenate(
        [packed[E_FAST:],
         jnp.full((NS * NCHS * CK - E_SLOW,), sentinel, jnp.int32)]
    ).reshape(NS, NCHS, CK)
    slow = jnp.concatenate(
        [slow, jnp.full((NS, NCHM - NCHS, CK), sentinel, jnp.int32)],
        axis=1)
    cores = [None, None]
    cores[FAST_C] = fast
    cores[1 - FAST_C] = slow
    epack = jnp.stack(cores, axis=1).reshape(NW, NCHM, CK)
    epack_sym = jnp.concatenate(
        [packed,
         jnp.full((NW * DNCH * CK - E_TOTAL,), sentinel, jnp.int32)]
    ).reshape(NW, DNCH, CK)
    batch3 = jnp.concatenate(
        [batch.astype(jnp.int32),
         jnp.full((NPAD - N_NODES,), NUM_GRAPHS, dtype=jnp.int32)]
    ).reshape(NW, NBCH, BK)
    x_pad = jnp.concatenate(
        [x.astype(f32), jnp.zeros((NPAD - N_NODES, EMB), f32)], axis=0)

    degp = _sc_degree(epack_sym)
    dinvb = _tc_dinv(degp)
    g = _tc_entry(x_pad, W[0], b[0].reshape(1, EMB), dinvb)

    h_final = None
    for l in range(NUM_LAYER):
        sp = _sc_scatter(g, epack)
        gam = gamma[l].reshape(1, EMB)
        bet = beta[l].reshape(1, EMB)
        last = l == NUM_LAYER - 1
        wn = W[0] if last else W[l + 1]
        bn = (b[0] if last else b[l + 1]).reshape(1, EMB)
        out = _tc_layer(sp, g, dinvb, gam, bet, wn, bn, last)
        if last:
            h_final = out
        else:
            g = out

    psum, cnt = _sc_pool(h_final, batch3)
    return _tc_pool_div(psum, cnt)
